# baseline (speedup 1.0000x reference)
import functools

import jax
import jax.numpy as jnp
from jax import lax
from jax.experimental import pallas as pl
from jax.experimental.pallas import tpu as pltpu
from jax.experimental.pallas import tpu_sc as plsc

N = 10000
E = 320000
D = 128
NC = 2
NS = 16
NW = NC * NS
C = 128
CH = 80
EPW = CH * C
E_PAD = NW * EPW
AGG_ROWS = 10112
ROWS_PER_W = AGG_ROWS // NS
PAD_DST = N + 8

RB = 1000
GRID = N // RB


def _gelu(x):
    return 0.5 * x * (1.0 + lax.erf(x * 0.7071067811865476))


def _segment_sum_sc(h, src_r, dst_r, zeros):
    mesh = plsc.VectorSubcoreMesh(
        core_axis_name="core", subcore_axis_name="subcore")

    @functools.partial(
        pl.kernel,
        out_type=jax.ShapeDtypeStruct((NC, AGG_ROWS, D), jnp.float32),
        mesh=mesh,
        scratch_types=[
            pltpu.VMEM((8, C), jnp.int32),
            pltpu.VMEM((8, C), jnp.int32),
            pltpu.VMEM((2, C, D), jnp.float32),
            pltpu.VMEM_SHARED((AGG_ROWS, D), jnp.float32),
            pltpu.SemaphoreType.DMA((2,)),
            pltpu.SemaphoreType.DMA((8,)),
            pltpu.SemaphoreType.DMA((8,)),
        ],
    )
    def seg_kernel(h_hbm, src_hbm, dst_hbm, z_hbm, out_hbm,
                   sidx, didx, rows, agg_sh, gsem, ssem, dsem):
        c = lax.axis_index("core")
        s = lax.axis_index("subcore")
        w = c * NS + s
        pltpu.sync_copy(z_hbm.at[pl.ds(s * ROWS_PER_W, ROWS_PER_W)],
                        agg_sh.at[pl.ds(s * ROWS_PER_W, ROWS_PER_W)])
        plsc.subcore_barrier()

        for t in range(8):
            pltpu.async_copy(src_hbm.at[w, t], sidx.at[t], ssem.at[t])
            pltpu.async_copy(dst_hbm.at[w, t], didx.at[t], dsem.at[t])
        for t in range(2):
            pltpu.make_async_copy(
                src_hbm.at[w, t], sidx.at[t], ssem.at[t]).wait()
            pltpu.async_copy(h_hbm.at[sidx.at[t]], rows.at[t], gsem.at[t])

        @pl.loop(0, CH, step=8)
        def _(j):
            for k in range(8):
                u = j + k
                k2, b = (k + 2) % 8, k % 2
                pltpu.make_async_copy(
                    h_hbm.at[sidx.at[k]], rows.at[b], gsem.at[b]).wait()
                pltpu.async_copy(src_hbm.at[w, u + 8], sidx.at[k], ssem.at[k])
                pltpu.make_async_copy(
                    dst_hbm.at[w, u], didx.at[k], dsem.at[k]).wait()
                pltpu.sync_copy(rows.at[b], agg_sh.at[didx.at[k]], add=True)
                pltpu.async_copy(dst_hbm.at[w, u + 8], didx.at[k], dsem.at[k])
                pltpu.make_async_copy(
                    src_hbm.at[w, u + 2], sidx.at[k2], ssem.at[k2]).wait()
                pltpu.async_copy(h_hbm.at[sidx.at[k2]], rows.at[b],
                                 gsem.at[b])

        for t in range(2):
            pltpu.make_async_copy(
                h_hbm.at[sidx.at[t]], rows.at[t], gsem.at[t]).wait()
        for t in range(2, 8):
            pltpu.make_async_copy(
                src_hbm.at[w, CH + t], sidx.at[t], ssem.at[t]).wait()
        for t in range(8):
            pltpu.make_async_copy(
                dst_hbm.at[w, CH + t], didx.at[t], dsem.at[t]).wait()
        plsc.subcore_barrier()
        pltpu.sync_copy(agg_sh.at[pl.ds(s * ROWS_PER_W, ROWS_PER_W)],
                        out_hbm.at[c, pl.ds(s * ROWS_PER_W, ROWS_PER_W)])

    return seg_kernel(h, src_r, dst_r, zeros)


def _enc_tc(x, Wt, b):
    def body(x_ref, w_ref, b_ref, o_ref):
        o_ref[...] = _gelu(
            jnp.dot(x_ref[...], w_ref[...],
                    preferred_element_type=jnp.float32) + b_ref[...])

    return pl.pallas_call(
        body,
        grid=(GRID,),
        in_specs=[
            pl.BlockSpec((RB, D), lambda i: (i, 0)),
            pl.BlockSpec((D, D), lambda i: (0, 0)),
            pl.BlockSpec((1, D), lambda i: (0, 0)),
        ],
        out_specs=pl.BlockSpec((RB, D), lambda i: (i, 0)),
        out_shape=jax.ShapeDtypeStruct((N, D), jnp.float32),
    )(x, Wt, b)


def _layer_tc(h, aggp, lsWt, lsb, lnWt, lnb):
    def body(h_ref, a_ref, lsw_ref, lsb_ref, lnw_ref, lnb_ref, o_ref):
        agg = a_ref[0] + a_ref[1]
        msg = jnp.dot(agg, lnw_ref[...],
                      preferred_element_type=jnp.float32) + lnb_ref[...]
        hs = jnp.dot(h_ref[...], lsw_ref[...],
                     preferred_element_type=jnp.float32) + lsb_ref[...]
        o_ref[...] = _gelu(hs + msg) + h_ref[...]

    return pl.pallas_call(
        body,
        grid=(GRID,),
        in_specs=[
            pl.BlockSpec((RB, D), lambda i: (i, 0)),
            pl.BlockSpec((NC, RB, D), lambda i: (0, i, 0)),
            pl.BlockSpec((D, D), lambda i: (0, 0)),
            pl.BlockSpec((1, D), lambda i: (0, 0)),
            pl.BlockSpec((D, D), lambda i: (0, 0)),
            pl.BlockSpec((1, D), lambda i: (0, 0)),
        ],
        out_specs=pl.BlockSpec((RB, D), lambda i: (i, 0)),
        out_shape=jax.ShapeDtypeStruct((N, D), jnp.float32),
    )(h, aggp, lsWt, lsb, lnWt, lnb)


def _dec_tc(h, Wt, b):
    def body(h_ref, w_ref, b_ref, o_ref):
        o_ref[...] = jnp.dot(h_ref[...], w_ref[...],
                             preferred_element_type=jnp.float32) + b_ref[...]

    return pl.pallas_call(
        body,
        grid=(GRID,),
        in_specs=[
            pl.BlockSpec((RB, D), lambda i: (i, 0)),
            pl.BlockSpec((D, D), lambda i: (0, 0)),
            pl.BlockSpec((1, D), lambda i: (0, 0)),
        ],
        out_specs=pl.BlockSpec((RB, D), lambda i: (i, 0)),
        out_shape=jax.ShapeDtypeStruct((N, D), jnp.float32),
    )(h, Wt, b)


def kernel(x, edge_index, enc_W, enc_b, ls_W, ls_b, ln_W, ln_b, dec_W, dec_b):
    pad = E_PAD - E
    src_r = jnp.concatenate(
        [edge_index[0], jnp.zeros((pad,), jnp.int32)]).reshape(NW, CH, C)
    dst_r = jnp.concatenate(
        [edge_index[1], jnp.full((pad,), PAD_DST, jnp.int32)]).reshape(NW, CH, C)
    src_r = jnp.concatenate(
        [src_r, jnp.zeros((NW, 8, C), jnp.int32)], axis=1)
    dst_r = jnp.concatenate(
        [dst_r, jnp.full((NW, 8, C), PAD_DST, jnp.int32)], axis=1)
    zeros = jnp.zeros((AGG_ROWS, D), jnp.float32)

    h = _enc_tc(x, enc_W.T, enc_b.reshape(1, D))
    for k in range(2):
        aggp = _segment_sum_sc(h, src_r, dst_r, zeros)
        h = _layer_tc(h, aggp, ls_W[k].T, ls_b[k].reshape(1, D),
                      ln_W[k].T, ln_b[k].reshape(1, D))
    return _dec_tc(h, dec_W.T, dec_b.reshape(1, D))


# speedup vs baseline: 1.8391x; 1.8391x over previous
import functools

import jax
import jax.numpy as jnp
from jax import lax
from jax.experimental import pallas as pl
from jax.experimental.pallas import tpu as pltpu
from jax.experimental.pallas import tpu_sc as plsc

N = 10000
E = 320000
D = 128
NC = 2
NS = 16
NW = NC * NS
C = 128
CH = 80
BLK = 16
NBLK = CH // BLK
EPW = CH * C
E_PAD = NW * EPW
AGG_ROWS = 10016
WIN = 640
PAD_DST = N + 8

RB = 1000
GRID = N // RB


def _gelu(x):
    return 0.5 * x * (1.0 + lax.erf(x * 0.7071067811865476))


def _segment_sum_sc(h, src_r, dst_r, zeros):
    mesh = plsc.VectorSubcoreMesh(
        core_axis_name="core", subcore_axis_name="subcore")

    @functools.partial(
        pl.kernel,
        out_type=jax.ShapeDtypeStruct((NC, AGG_ROWS, D), jnp.float32),
        mesh=mesh,
        scratch_types=[
            pltpu.VMEM((BLK, C), jnp.int32),
            pltpu.VMEM((BLK, C), jnp.int32),
            pltpu.VMEM((2, C, D), jnp.float32),
            pltpu.VMEM_SHARED((AGG_ROWS, D), jnp.float32),
            pltpu.SemaphoreType.DMA((2,)),
        ],
    )
    def seg_kernel(h_hbm, src_hbm, dst_hbm, z_hbm, out_hbm,
                   srcb, dstb, rows, agg_sh, gsem):
        c = lax.axis_index("core")
        s = lax.axis_index("subcore")
        w = c * NS + s
        base = pl.multiple_of(jnp.minimum(s * WIN, AGG_ROWS - WIN), 8)
        pltpu.sync_copy(z_hbm.at[pl.ds(base, WIN)],
                        agg_sh.at[pl.ds(base, WIN)])
        plsc.subcore_barrier()

        @pl.loop(0, NBLK)
        def _(blk):
            pltpu.sync_copy(src_hbm.at[w].at[pl.ds(blk * BLK, BLK)], srcb)
            pltpu.sync_copy(dst_hbm.at[w].at[pl.ds(blk * BLK, BLK)], dstb)
            pltpu.async_copy(h_hbm.at[srcb.at[0]], rows.at[0], gsem.at[0])
            pltpu.async_copy(h_hbm.at[srcb.at[1]], rows.at[1], gsem.at[1])
            for k in range(BLK):
                b = k % 2
                pltpu.make_async_copy(
                    h_hbm.at[srcb.at[k]], rows.at[b], gsem.at[b]).wait()
                pltpu.sync_copy(rows.at[b], agg_sh.at[dstb.at[k]], add=True)
                if k + 2 < BLK:
                    pltpu.async_copy(
                        h_hbm.at[srcb.at[k + 2]], rows.at[b], gsem.at[b])

        plsc.subcore_barrier()
        pltpu.sync_copy(agg_sh.at[pl.ds(base, WIN)],
                        out_hbm.at[c, pl.ds(base, WIN)])

    return seg_kernel(h, src_r, dst_r, zeros)


def _enc_tc(x, Wt, b):
    def body(x_ref, w_ref, b_ref, o_ref):
        o_ref[...] = _gelu(
            jnp.dot(x_ref[...], w_ref[...],
                    preferred_element_type=jnp.float32) + b_ref[...])

    return pl.pallas_call(
        body,
        grid=(GRID,),
        in_specs=[
            pl.BlockSpec((RB, D), lambda i: (i, 0)),
            pl.BlockSpec((D, D), lambda i: (0, 0)),
            pl.BlockSpec((1, D), lambda i: (0, 0)),
        ],
        out_specs=pl.BlockSpec((RB, D), lambda i: (i, 0)),
        out_shape=jax.ShapeDtypeStruct((N, D), jnp.float32),
    )(x, Wt, b)


def _layer_tc(h, aggp, lsWt, lsb, lnWt, lnb):
    def body(h_ref, a_ref, lsw_ref, lsb_ref, lnw_ref, lnb_ref, o_ref):
        agg = a_ref[0] + a_ref[1]
        msg = jnp.dot(agg, lnw_ref[...],
                      preferred_element_type=jnp.float32) + lnb_ref[...]
        hs = jnp.dot(h_ref[...], lsw_ref[...],
                     preferred_element_type=jnp.float32) + lsb_ref[...]
        o_ref[...] = _gelu(hs + msg) + h_ref[...]

    return pl.pallas_call(
        body,
        grid=(GRID,),
        in_specs=[
            pl.BlockSpec((RB, D), lambda i: (i, 0)),
            pl.BlockSpec((NC, RB, D), lambda i: (0, i, 0)),
            pl.BlockSpec((D, D), lambda i: (0, 0)),
            pl.BlockSpec((1, D), lambda i: (0, 0)),
            pl.BlockSpec((D, D), lambda i: (0, 0)),
            pl.BlockSpec((1, D), lambda i: (0, 0)),
        ],
        out_specs=pl.BlockSpec((RB, D), lambda i: (i, 0)),
        out_shape=jax.ShapeDtypeStruct((N, D), jnp.float32),
    )(h, aggp, lsWt, lsb, lnWt, lnb)


def _dec_tc(h, Wt, b):
    def body(h_ref, w_ref, b_ref, o_ref):
        o_ref[...] = jnp.dot(h_ref[...], w_ref[...],
                             preferred_element_type=jnp.float32) + b_ref[...]

    return pl.pallas_call(
        body,
        grid=(GRID,),
        in_specs=[
            pl.BlockSpec((RB, D), lambda i: (i, 0)),
            pl.BlockSpec((D, D), lambda i: (0, 0)),
            pl.BlockSpec((1, D), lambda i: (0, 0)),
        ],
        out_specs=pl.BlockSpec((RB, D), lambda i: (i, 0)),
        out_shape=jax.ShapeDtypeStruct((N, D), jnp.float32),
    )(h, Wt, b)


def kernel(x, edge_index, enc_W, enc_b, ls_W, ls_b, ln_W, ln_b, dec_W, dec_b):
    pad = E_PAD - E
    src_r = jnp.concatenate(
        [edge_index[0], jnp.zeros((pad,), jnp.int32)]).reshape(NW, CH, C)
    dst_r = jnp.concatenate(
        [edge_index[1], jnp.full((pad,), PAD_DST, jnp.int32)]).reshape(NW, CH, C)
    zeros = jnp.zeros((AGG_ROWS, D), jnp.float32)

    h = _enc_tc(x, enc_W.T, enc_b.reshape(1, D))
    for k in range(2):
        aggp = _segment_sum_sc(h, src_r, dst_r, zeros)
        h = _layer_tc(h, aggp, ls_W[k].T, ls_b[k].reshape(1, D),
                      ln_W[k].T, ln_b[k].reshape(1, D))
    return _dec_tc(h, dec_W.T, dec_b.reshape(1, D))


# speedup vs baseline: 1.8427x; 1.0019x over previous
import functools

import jax
import jax.numpy as jnp
from jax import lax
from jax.experimental import pallas as pl
from jax.experimental.pallas import tpu as pltpu
from jax.experimental.pallas import tpu_sc as plsc

N = 10000
E = 320000
D = 128
NC = 2
NS = 16
NW = NC * NS
C = 128
CH = 80
BLK = 16
NBLK = CH // BLK
EPW = CH * C
E_PAD = NW * EPW
AGG_ROWS = 10016
WIN = 640
PAD_DST = N + 8

RB = 1000
GRID = N // RB


def _gelu(x):
    return 0.5 * x * (1.0 + lax.erf(x * 0.7071067811865476))


def _segment_sum_sc(h, src_r, dst_r, zeros):
    mesh = plsc.VectorSubcoreMesh(
        core_axis_name="core", subcore_axis_name="subcore")

    @functools.partial(
        pl.kernel,
        out_type=jax.ShapeDtypeStruct((NC, AGG_ROWS, D), jnp.float32),
        mesh=mesh,
        scratch_types=[
            pltpu.VMEM((BLK, C), jnp.int32),
            pltpu.VMEM((BLK, C), jnp.int32),
            pltpu.VMEM((2, C, D), jnp.float32),
            pltpu.VMEM_SHARED((AGG_ROWS, D), jnp.float32),
            pltpu.SemaphoreType.DMA((2,)),
            pltpu.SemaphoreType.DMA((2,)),
        ],
    )
    def seg_kernel(h_hbm, src_hbm, dst_hbm, z_hbm, out_hbm,
                   srcb, dstb, rows, agg_sh, gsem, ssem):
        c = lax.axis_index("core")
        s = lax.axis_index("subcore")
        w = c * NS + s
        base = pl.multiple_of(jnp.minimum(s * WIN, AGG_ROWS - WIN), 8)
        pltpu.sync_copy(z_hbm.at[pl.ds(base, WIN)],
                        agg_sh.at[pl.ds(base, WIN)])
        plsc.subcore_barrier()

        @pl.loop(0, NBLK)
        def _(blk):
            pltpu.sync_copy(src_hbm.at[w].at[pl.ds(blk * BLK, BLK)], srcb)
            pltpu.sync_copy(dst_hbm.at[w].at[pl.ds(blk * BLK, BLK)], dstb)
            pltpu.async_copy(h_hbm.at[srcb.at[0]], rows.at[0], gsem.at[0])
            pltpu.async_copy(h_hbm.at[srcb.at[1]], rows.at[1], gsem.at[1])
            for k in range(BLK):
                b = k % 2
                pltpu.make_async_copy(
                    h_hbm.at[srcb.at[k]], rows.at[b], gsem.at[b]).wait()
                if k >= 2:
                    pltpu.make_async_copy(
                        rows.at[b], agg_sh.at[dstb.at[k - 2]],
                        ssem.at[b]).wait()
                pltpu.async_copy(rows.at[b], agg_sh.at[dstb.at[k]],
                                 ssem.at[b], add=True)
                if k + 2 < BLK:
                    pltpu.async_copy(
                        h_hbm.at[srcb.at[k + 2]], rows.at[b], gsem.at[b])
            pltpu.make_async_copy(
                rows.at[0], agg_sh.at[dstb.at[BLK - 2]], ssem.at[0]).wait()
            pltpu.make_async_copy(
                rows.at[1], agg_sh.at[dstb.at[BLK - 1]], ssem.at[1]).wait()

        plsc.subcore_barrier()
        pltpu.sync_copy(agg_sh.at[pl.ds(base, WIN)],
                        out_hbm.at[c, pl.ds(base, WIN)])

    return seg_kernel(h, src_r, dst_r, zeros)


def _enc_tc(x, Wt, b):
    def body(x_ref, w_ref, b_ref, o_ref):
        o_ref[...] = _gelu(
            jnp.dot(x_ref[...], w_ref[...],
                    preferred_element_type=jnp.float32) + b_ref[...])

    return pl.pallas_call(
        body,
        grid=(GRID,),
        in_specs=[
            pl.BlockSpec((RB, D), lambda i: (i, 0)),
            pl.BlockSpec((D, D), lambda i: (0, 0)),
            pl.BlockSpec((1, D), lambda i: (0, 0)),
        ],
        out_specs=pl.BlockSpec((RB, D), lambda i: (i, 0)),
        out_shape=jax.ShapeDtypeStruct((N, D), jnp.float32),
    )(x, Wt, b)


def _layer_tc(h, aggp, lsWt, lsb, lnWt, lnb):
    def body(h_ref, a_ref, lsw_ref, lsb_ref, lnw_ref, lnb_ref, o_ref):
        agg = a_ref[0] + a_ref[1]
        msg = jnp.dot(agg, lnw_ref[...],
                      preferred_element_type=jnp.float32) + lnb_ref[...]
        hs = jnp.dot(h_ref[...], lsw_ref[...],
                     preferred_element_type=jnp.float32) + lsb_ref[...]
        o_ref[...] = _gelu(hs + msg) + h_ref[...]

    return pl.pallas_call(
        body,
        grid=(GRID,),
        in_specs=[
            pl.BlockSpec((RB, D), lambda i: (i, 0)),
            pl.BlockSpec((NC, RB, D), lambda i: (0, i, 0)),
            pl.BlockSpec((D, D), lambda i: (0, 0)),
            pl.BlockSpec((1, D), lambda i: (0, 0)),
            pl.BlockSpec((D, D), lambda i: (0, 0)),
            pl.BlockSpec((1, D), lambda i: (0, 0)),
        ],
        out_specs=pl.BlockSpec((RB, D), lambda i: (i, 0)),
        out_shape=jax.ShapeDtypeStruct((N, D), jnp.float32),
    )(h, aggp, lsWt, lsb, lnWt, lnb)


def _dec_tc(h, Wt, b):
    def body(h_ref, w_ref, b_ref, o_ref):
        o_ref[...] = jnp.dot(h_ref[...], w_ref[...],
                             preferred_element_type=jnp.float32) + b_ref[...]

    return pl.pallas_call(
        body,
        grid=(GRID,),
        in_specs=[
            pl.BlockSpec((RB, D), lambda i: (i, 0)),
            pl.BlockSpec((D, D), lambda i: (0, 0)),
            pl.BlockSpec((1, D), lambda i: (0, 0)),
        ],
        out_specs=pl.BlockSpec((RB, D), lambda i: (i, 0)),
        out_shape=jax.ShapeDtypeStruct((N, D), jnp.float32),
    )(h, Wt, b)


def kernel(x, edge_index, enc_W, enc_b, ls_W, ls_b, ln_W, ln_b, dec_W, dec_b):
    pad = E_PAD - E
    src_r = jnp.concatenate(
        [edge_index[0], jnp.zeros((pad,), jnp.int32)]).reshape(NW, CH, C)
    dst_r = jnp.concatenate(
        [edge_index[1], jnp.full((pad,), PAD_DST, jnp.int32)]).reshape(NW, CH, C)
    zeros = jnp.zeros((AGG_ROWS, D), jnp.float32)

    h = _enc_tc(x, enc_W.T, enc_b.reshape(1, D))
    for k in range(2):
        aggp = _segment_sum_sc(h, src_r, dst_r, zeros)
        h = _layer_tc(h, aggp, ls_W[k].T, ls_b[k].reshape(1, D),
                      ln_W[k].T, ln_b[k].reshape(1, D))
    return _dec_tc(h, dec_W.T, dec_b.reshape(1, D))


# speedup vs baseline: 1.8455x; 1.0016x over previous
import functools

import jax
import jax.numpy as jnp
from jax import lax
from jax.experimental import pallas as pl
from jax.experimental.pallas import tpu as pltpu
from jax.experimental.pallas import tpu_sc as plsc

N = 10000
E = 320000
D = 128
NC = 2
NS = 16
NW = NC * NS
C = 128
CH = 80
BLK = 16
NBLK = CH // BLK
EPW = CH * C
E_PAD = NW * EPW
AGG_ROWS = 10016
WIN = 640
PAD_DST = N + 8

RB = 1000
GRID = N // RB


def _gelu(x):
    return 0.5 * x * (1.0 + lax.erf(x * 0.7071067811865476))


def _segment_sum_sc(h, src_r, dst_r, zeros):
    mesh = plsc.VectorSubcoreMesh(
        core_axis_name="core", subcore_axis_name="subcore")

    @functools.partial(
        pl.kernel,
        out_type=jax.ShapeDtypeStruct((NC, AGG_ROWS, D), jnp.float32),
        mesh=mesh,
        scratch_types=[
            pltpu.VMEM((BLK, C), jnp.int32),
            pltpu.VMEM((BLK, C), jnp.int32),
            pltpu.VMEM((2, C, D), jnp.float32),
            pltpu.VMEM_SHARED((AGG_ROWS, D), jnp.float32),
            pltpu.SemaphoreType.DMA((2,)),
        ],
    )
    def seg_kernel(h_hbm, src_hbm, dst_hbm, z_hbm, out_hbm,
                   srcb, dstb, rows, agg_sh, gsem):
        c = lax.axis_index("core")
        s = lax.axis_index("subcore")
        w = c * NS + s
        base = pl.multiple_of(jnp.minimum(s * WIN, AGG_ROWS - WIN), 8)
        pltpu.sync_copy(z_hbm.at[pl.ds(base, WIN)],
                        agg_sh.at[pl.ds(base, WIN)])
        plsc.subcore_barrier()

        @pl.loop(0, NBLK)
        def _(blk):
            pltpu.sync_copy(src_hbm.at[w].at[pl.ds(blk * BLK, BLK)], srcb)
            pltpu.sync_copy(dst_hbm.at[w].at[pl.ds(blk * BLK, BLK)], dstb)
            pltpu.async_copy(h_hbm.at[srcb.at[0]], rows.at[0], gsem.at[0])
            pltpu.async_copy(h_hbm.at[srcb.at[1]], rows.at[1], gsem.at[1])
            for k in range(BLK):
                b = k % 2
                pltpu.make_async_copy(
                    h_hbm.at[srcb.at[k]], rows.at[b], gsem.at[b]).wait()
                pltpu.sync_copy(rows.at[b], agg_sh.at[dstb.at[k]], add=True)
                if k + 2 < BLK:
                    pltpu.async_copy(
                        h_hbm.at[srcb.at[k + 2]], rows.at[b], gsem.at[b])

        plsc.subcore_barrier()
        pltpu.sync_copy(agg_sh.at[pl.ds(base, WIN)],
                        out_hbm.at[c, pl.ds(base, WIN)])

    return seg_kernel(h, src_r, dst_r, zeros)


def _enc_tc(x, Wt, b):
    def body(x_ref, w_ref, b_ref, o_ref):
        o_ref[...] = _gelu(
            jnp.dot(x_ref[...], w_ref[...],
                    preferred_element_type=jnp.float32) + b_ref[...])

    return pl.pallas_call(
        body,
        grid=(GRID,),
        in_specs=[
            pl.BlockSpec((RB, D), lambda i: (i, 0)),
            pl.BlockSpec((D, D), lambda i: (0, 0)),
            pl.BlockSpec((1, D), lambda i: (0, 0)),
        ],
        out_specs=pl.BlockSpec((RB, D), lambda i: (i, 0)),
        out_shape=jax.ShapeDtypeStruct((N, D), jnp.float32),
    )(x, Wt, b)


def _layer_tc(h, aggp, lsWt, lsb, lnWt, lnb):
    def body(h_ref, a_ref, lsw_ref, lsb_ref, lnw_ref, lnb_ref, o_ref):
        agg = a_ref[0] + a_ref[1]
        msg = jnp.dot(agg, lnw_ref[...],
                      preferred_element_type=jnp.float32) + lnb_ref[...]
        hs = jnp.dot(h_ref[...], lsw_ref[...],
                     preferred_element_type=jnp.float32) + lsb_ref[...]
        o_ref[...] = _gelu(hs + msg) + h_ref[...]

    return pl.pallas_call(
        body,
        grid=(GRID,),
        in_specs=[
            pl.BlockSpec((RB, D), lambda i: (i, 0)),
            pl.BlockSpec((NC, RB, D), lambda i: (0, i, 0)),
            pl.BlockSpec((D, D), lambda i: (0, 0)),
            pl.BlockSpec((1, D), lambda i: (0, 0)),
            pl.BlockSpec((D, D), lambda i: (0, 0)),
            pl.BlockSpec((1, D), lambda i: (0, 0)),
        ],
        out_specs=pl.BlockSpec((RB, D), lambda i: (i, 0)),
        out_shape=jax.ShapeDtypeStruct((N, D), jnp.float32),
    )(h, aggp, lsWt, lsb, lnWt, lnb)


def _layer_dec_tc(h, aggp, lsWt, lsb, lnWt, lnb, decWt, decb):
    def body(h_ref, a_ref, lsw_ref, lsb_ref, lnw_ref, lnb_ref,
             dw_ref, db_ref, o_ref):
        agg = a_ref[0] + a_ref[1]
        msg = jnp.dot(agg, lnw_ref[...],
                      preferred_element_type=jnp.float32) + lnb_ref[...]
        hs = jnp.dot(h_ref[...], lsw_ref[...],
                     preferred_element_type=jnp.float32) + lsb_ref[...]
        hn = _gelu(hs + msg) + h_ref[...]
        o_ref[...] = jnp.dot(hn, dw_ref[...],
                             preferred_element_type=jnp.float32) + db_ref[...]

    return pl.pallas_call(
        body,
        grid=(GRID,),
        in_specs=[
            pl.BlockSpec((RB, D), lambda i: (i, 0)),
            pl.BlockSpec((NC, RB, D), lambda i: (0, i, 0)),
            pl.BlockSpec((D, D), lambda i: (0, 0)),
            pl.BlockSpec((1, D), lambda i: (0, 0)),
            pl.BlockSpec((D, D), lambda i: (0, 0)),
            pl.BlockSpec((1, D), lambda i: (0, 0)),
            pl.BlockSpec((D, D), lambda i: (0, 0)),
            pl.BlockSpec((1, D), lambda i: (0, 0)),
        ],
        out_specs=pl.BlockSpec((RB, D), lambda i: (i, 0)),
        out_shape=jax.ShapeDtypeStruct((N, D), jnp.float32),
    )(h, aggp, lsWt, lsb, lnWt, lnb, decWt, decb)


def _dec_tc(h, Wt, b):
    def body(h_ref, w_ref, b_ref, o_ref):
        o_ref[...] = jnp.dot(h_ref[...], w_ref[...],
                             preferred_element_type=jnp.float32) + b_ref[...]

    return pl.pallas_call(
        body,
        grid=(GRID,),
        in_specs=[
            pl.BlockSpec((RB, D), lambda i: (i, 0)),
            pl.BlockSpec((D, D), lambda i: (0, 0)),
            pl.BlockSpec((1, D), lambda i: (0, 0)),
        ],
        out_specs=pl.BlockSpec((RB, D), lambda i: (i, 0)),
        out_shape=jax.ShapeDtypeStruct((N, D), jnp.float32),
    )(h, Wt, b)


def kernel(x, edge_index, enc_W, enc_b, ls_W, ls_b, ln_W, ln_b, dec_W, dec_b):
    pad = E_PAD - E
    src_r = jnp.concatenate(
        [edge_index[0], jnp.zeros((pad,), jnp.int32)]).reshape(NW, CH, C)
    dst_r = jnp.concatenate(
        [edge_index[1], jnp.full((pad,), PAD_DST, jnp.int32)]).reshape(NW, CH, C)
    zeros = jnp.zeros((AGG_ROWS, D), jnp.float32)

    h = _enc_tc(x, enc_W.T, enc_b.reshape(1, D))
    aggp = _segment_sum_sc(h, src_r, dst_r, zeros)
    h = _layer_tc(h, aggp, ls_W[0].T, ls_b[0].reshape(1, D),
                  ln_W[0].T, ln_b[0].reshape(1, D))
    aggp = _segment_sum_sc(h, src_r, dst_r, zeros)
    return _layer_dec_tc(h, aggp, ls_W[1].T, ls_b[1].reshape(1, D),
                         ln_W[1].T, ln_b[1].reshape(1, D),
                         dec_W.T, dec_b.reshape(1, D))
